# baseline (speedup 1.0000x reference)
import functools

import jax
import jax.numpy as jnp
from jax import lax
from jax.experimental import pallas as pl
from jax.experimental.pallas import tpu as pltpu
from jax.experimental.pallas import tpu_sc as plsc

N = 10000
E = 320000
H = 128
NC, NS = 2, 16
NW = NC * NS
BLK = 128
NPAD = 10240
RPW = NPAD // NW
KMAX = 96
CAPB = 192
CAP = CAPB * BLK
NEXT = N + 128
LROWS = NS * RPW
LPAD = LROWS + 256

DBLKS = 80
DEPW = DBLKS * BLK
DEPAD = NW * DEPW


def _agg_body(h_hbm, srcs_hbm, dsts_hbm, zrow_hbm, out_hbm,
              src_v, dst_v, rows_v, acc_s, sem):
    c = lax.axis_index("c")
    s = lax.axis_index("s")
    w = s * NC + c
    pltpu.sync_copy(srcs_hbm.at[w], src_v)
    pltpu.sync_copy(dsts_hbm.at[w], dst_v)
    pltpu.sync_copy(zrow_hbm, acc_s.at[pl.ds(s * RPW, RPW)])

    @pl.loop(0, CAPB)
    def _(j):
        pltpu.async_copy(h_hbm.at[src_v.at[j]], rows_v, sem).wait()
        pltpu.sync_copy(rows_v, acc_s.at[dst_v.at[j]], add=True)

    pltpu.sync_copy(acc_s.at[pl.ds(s * RPW, RPW)],
                    out_hbm.at[pl.ds(w * RPW, RPW)])


def _deg_body(dsts_hbm, ones_hbm, zvec_hbm, out_hbm, dst_v, ones_v, deg_s):
    c = lax.axis_index("c")
    s = lax.axis_index("s")
    w = s * NC + c
    pltpu.sync_copy(dsts_hbm.at[w], dst_v)
    pltpu.sync_copy(ones_hbm, ones_v)
    pltpu.sync_copy(zvec_hbm, deg_s.at[pl.ds(s * (NPAD // NS), NPAD // NS)])
    plsc.subcore_barrier()

    @pl.loop(0, DBLKS)
    def _(j):
        pltpu.sync_copy(ones_v, deg_s.at[dst_v.at[j]], add=True)

    plsc.subcore_barrier()
    pltpu.sync_copy(deg_s.at[pl.ds(s * (NPAD // NS), NPAD // NS)],
                    out_hbm.at[c, pl.ds(s * (NPAD // NS), NPAD // NS)])


@functools.cache
def _sc_kernels():
    mesh = plsc.VectorSubcoreMesh(core_axis_name="c", subcore_axis_name="s")
    agg = pl.kernel(
        _agg_body,
        mesh=mesh,
        out_type=jax.ShapeDtypeStruct((NPAD, H), jnp.float32),
        scratch_types=[
            pltpu.VMEM((CAPB, BLK), jnp.int32),
            pltpu.VMEM((CAPB, BLK), jnp.int32),
            pltpu.VMEM((BLK, H), jnp.float32),
            pltpu.VMEM_SHARED((LPAD, H), jnp.float32),
            pltpu.SemaphoreType.DMA,
        ],
    )
    deg = pl.kernel(
        _deg_body,
        mesh=mesh,
        out_type=jax.ShapeDtypeStruct((NC, NPAD), jnp.float32),
        scratch_types=[
            pltpu.VMEM((DBLKS, BLK), jnp.int32),
            pltpu.VMEM((BLK,), jnp.float32),
            pltpu.VMEM_SHARED((NPAD,), jnp.float32),
        ],
    )
    return agg, deg


def _stage1(x_ref, w_ref, b_ref, o_ref):
    h = jnp.dot(x_ref[...], w_ref[...], preferred_element_type=jnp.float32)
    o_ref[...] = jnp.maximum(h + b_ref[...], 0.0)


def _stage_mid(a_ref, r_ref, w_ref, b_ref, o_ref):
    h = a_ref[0:N, :] * r_ref[...]
    h = jnp.dot(h, w_ref[...], preferred_element_type=jnp.float32)
    o_ref[...] = jnp.maximum(h + b_ref[...], 0.0)


def _norm_pad(h_ref, m_ref, v_ref, g_ref, be_ref, o_ref):
    o_ref[0:N, :] = ((h_ref[...] - m_ref[...]) / jnp.sqrt(v_ref[...] + 1e-5)
                     * g_ref[...] + be_ref[...])
    o_ref[N:NEXT, :] = jnp.zeros((NEXT - N, H), jnp.float32)


def _norm(h_ref, m_ref, v_ref, g_ref, be_ref, o_ref):
    o_ref[...] = ((h_ref[...] - m_ref[...]) / jnp.sqrt(v_ref[...] + 1e-5)
                  * g_ref[...] + be_ref[...])


def _pcall(body, rows):
    return pl.pallas_call(
        body, out_shape=jax.ShapeDtypeStruct((rows, H), jnp.float32))


def kernel(x, edge_index, W1, b1, W2, b2, W3, b3, gamma, beta, Wr, br, Wk, bk):
    src, dst = edge_index[0], edge_index[1]

    order = jnp.argsort(dst, stable=True)
    src_s = src[order]
    dst_s = dst[order]
    iE = jnp.arange(E, dtype=jnp.int32)
    occ = iE - jnp.searchsorted(dst_s, dst_s, side="left").astype(jnp.int32)
    occ = jnp.minimum(occ, KMAX - 1)
    wk = dst_s // RPW
    seg = wk * KMAX + occ
    key = seg * NPAD + dst_s
    ord2 = jnp.argsort(key)
    src_l, dst_l, seg_l = src_s[ord2], dst_s[ord2], seg[ord2]
    cnts = jnp.bincount(seg, length=NW * KMAX).astype(jnp.int32)
    padded = ((cnts + BLK - 1) // BLK) * BLK
    offs = (jnp.cumsum(padded.reshape(NW, KMAX), axis=1)
            - padded.reshape(NW, KMAX)).reshape(-1)
    pos_in = iE - jnp.searchsorted(seg_l, seg_l, side="left").astype(jnp.int32)
    slot = (seg_l // KMAX) * CAP + offs[seg_l] + pos_in
    all_slots = jnp.arange(NW * CAP, dtype=jnp.int32)
    pad_src = N + (all_slots % (NEXT - N))
    pad_dst = LROWS + (all_slots % (LPAD - LROWS))
    ldst = dst_l - (wk[ord2] - wk[ord2] // 2) * RPW
    srcs = pad_src.at[slot].set(src_l).reshape(NW, CAPB, BLK)
    dsts = pad_dst.at[slot].set(ldst).reshape(NW, CAPB, BLK)

    pid = jnp.arange(DEPAD - E, dtype=jnp.int32)
    dsts_deg = jnp.concatenate([dst, N + pid % (NPAD - N)]).reshape(
        NW, DBLKS, BLK)

    zrow = jnp.zeros((RPW, H), jnp.float32)
    zvec = jnp.zeros((NPAD // NS,), jnp.float32)
    ones = jnp.ones((BLK,), jnp.float32)
    b1r, b2r, b3r = (b.reshape(1, H) for b in (b1, b2, b3))
    gr, ber = gamma.reshape(1, H), beta.reshape(1, H)
    W1s = W1[: x.shape[1]]

    _agg_sc, _deg_sc = _sc_kernels()

    degp = _deg_sc(dsts_deg, ones, zvec)
    deg = degp[0, :N] + degp[1, :N]
    r = (1.0 / jnp.maximum(deg, 1.0)).reshape(N, 1)

    def stats1(xin, W, b):
        h = jax.nn.relu(xin @ W + b)
        m = h.mean(axis=0)
        v = ((h - m) ** 2).mean(axis=0)
        return m.reshape(1, H), v.reshape(1, H)

    def stats_mid(a, W, b):
        h = jax.nn.relu((a[:N] * r) @ W + b)
        m = h.mean(axis=0)
        v = ((h - m) ** 2).mean(axis=0)
        return m.reshape(1, H), v.reshape(1, H)

    hp1 = _pcall(_stage1, N)(x, W1s, b1r)
    m1, v1 = stats1(x, W1s, b1)
    h1 = _pcall(_norm_pad, NEXT)(hp1, m1, v1, gr, ber)
    a1 = _agg_sc(h1, srcs, dsts, zrow)

    hp2 = _pcall(_stage_mid, N)(a1, r, W2, b2r)
    m2, v2 = stats_mid(a1, W2, b2)
    h2 = _pcall(_norm_pad, NEXT)(hp2, m2, v2, gr, ber)
    a2 = _agg_sc(h2, srcs, dsts, zrow)

    hp3 = _pcall(_stage_mid, N)(a2, r, W3, b3r)
    m3, v3 = stats_mid(a2, W3, b3)
    h3 = _pcall(_norm, N)(hp3, m3, v3, gr, ber)

    pooled = h3.mean(axis=0, keepdims=True)
    regime_logits = pooled @ Wr + br
    risk = jax.nn.sigmoid(pooled @ Wk + bk)
    return (regime_logits, risk, h3)


# speedup vs baseline: 1.0060x; 1.0060x over previous
import functools

import jax
import jax.numpy as jnp
from jax import lax
from jax.experimental import pallas as pl
from jax.experimental.pallas import tpu as pltpu
from jax.experimental.pallas import tpu_sc as plsc

N = 10000
E = 320000
H = 128
NC, NS = 2, 16
NW = NC * NS
BLK = 128
NPAD = 10240
RPW = NPAD // NW
KMAX = 96
CAPB = 192
CAP = CAPB * BLK
NEXT = N + 128
LROWS = NS * RPW
LPAD = LROWS + 256

DBLKS = 80
DEPW = DBLKS * BLK
DEPAD = NW * DEPW


def _agg_body(h_hbm, srcs_hbm, dsts_hbm, zrow_hbm, out_hbm,
              src_v, dst_v, rows0, rows1, acc_s, sem0, sem1):
    c = lax.axis_index("c")
    s = lax.axis_index("s")
    w = s * NC + c
    pltpu.sync_copy(srcs_hbm.at[w], src_v)
    pltpu.sync_copy(dsts_hbm.at[w], dst_v)
    pltpu.sync_copy(zrow_hbm, acc_s.at[pl.ds(s * RPW, RPW)])

    pltpu.async_copy(h_hbm.at[src_v.at[0]], rows0, sem0)

    @pl.loop(0, CAPB, step=2)
    def _(j):
        pltpu.make_async_copy(h_hbm.at[src_v.at[j]], rows0, sem0).wait()
        pltpu.async_copy(h_hbm.at[src_v.at[j + 1]], rows1, sem1)
        pltpu.sync_copy(rows0, acc_s.at[dst_v.at[j]], add=True)
        pltpu.make_async_copy(h_hbm.at[src_v.at[j + 1]], rows1, sem1).wait()
        j2 = jnp.minimum(j + 2, CAPB - 1)
        pltpu.async_copy(h_hbm.at[src_v.at[j2]], rows0, sem0)
        pltpu.sync_copy(rows1, acc_s.at[dst_v.at[j + 1]], add=True)

    pltpu.make_async_copy(h_hbm.at[src_v.at[CAPB - 1]], rows0, sem0).wait()
    pltpu.sync_copy(acc_s.at[pl.ds(s * RPW, RPW)],
                    out_hbm.at[pl.ds(w * RPW, RPW)])


def _deg_body(dsts_hbm, ones_hbm, zvec_hbm, out_hbm, dst_v, ones_v, deg_s):
    c = lax.axis_index("c")
    s = lax.axis_index("s")
    w = s * NC + c
    pltpu.sync_copy(dsts_hbm.at[w], dst_v)
    pltpu.sync_copy(ones_hbm, ones_v)
    pltpu.sync_copy(zvec_hbm, deg_s.at[pl.ds(s * (NPAD // NS), NPAD // NS)])
    plsc.subcore_barrier()

    @pl.loop(0, DBLKS)
    def _(j):
        pltpu.sync_copy(ones_v, deg_s.at[dst_v.at[j]], add=True)

    plsc.subcore_barrier()
    pltpu.sync_copy(deg_s.at[pl.ds(s * (NPAD // NS), NPAD // NS)],
                    out_hbm.at[c, pl.ds(s * (NPAD // NS), NPAD // NS)])


@functools.cache
def _sc_kernels():
    mesh = plsc.VectorSubcoreMesh(core_axis_name="c", subcore_axis_name="s")
    agg = pl.kernel(
        _agg_body,
        mesh=mesh,
        out_type=jax.ShapeDtypeStruct((NPAD, H), jnp.float32),
        scratch_types=[
            pltpu.VMEM((CAPB, BLK), jnp.int32),
            pltpu.VMEM((CAPB, BLK), jnp.int32),
            pltpu.VMEM((BLK, H), jnp.float32),
            pltpu.VMEM((BLK, H), jnp.float32),
            pltpu.VMEM_SHARED((LPAD, H), jnp.float32),
            pltpu.SemaphoreType.DMA,
            pltpu.SemaphoreType.DMA,
        ],
    )
    deg = pl.kernel(
        _deg_body,
        mesh=mesh,
        out_type=jax.ShapeDtypeStruct((NC, NPAD), jnp.float32),
        scratch_types=[
            pltpu.VMEM((DBLKS, BLK), jnp.int32),
            pltpu.VMEM((BLK,), jnp.float32),
            pltpu.VMEM_SHARED((NPAD,), jnp.float32),
        ],
    )
    return agg, deg


def _stage1(x_ref, w_ref, b_ref, o_ref):
    h = jnp.dot(x_ref[...], w_ref[...], preferred_element_type=jnp.float32)
    o_ref[...] = jnp.maximum(h + b_ref[...], 0.0)


def _stage_mid(a_ref, r_ref, w_ref, b_ref, o_ref):
    h = a_ref[0:N, :] * r_ref[...]
    h = jnp.dot(h, w_ref[...], preferred_element_type=jnp.float32)
    o_ref[...] = jnp.maximum(h + b_ref[...], 0.0)


def _norm_pad(h_ref, m_ref, v_ref, g_ref, be_ref, o_ref):
    o_ref[0:N, :] = ((h_ref[...] - m_ref[...]) / jnp.sqrt(v_ref[...] + 1e-5)
                     * g_ref[...] + be_ref[...])
    o_ref[N:NEXT, :] = jnp.zeros((NEXT - N, H), jnp.float32)


def _norm(h_ref, m_ref, v_ref, g_ref, be_ref, o_ref):
    o_ref[...] = ((h_ref[...] - m_ref[...]) / jnp.sqrt(v_ref[...] + 1e-5)
                  * g_ref[...] + be_ref[...])


def _pcall(body, rows):
    return pl.pallas_call(
        body, out_shape=jax.ShapeDtypeStruct((rows, H), jnp.float32))


def kernel(x, edge_index, W1, b1, W2, b2, W3, b3, gamma, beta, Wr, br, Wk, bk):
    src, dst = edge_index[0], edge_index[1]

    order = jnp.argsort(dst, stable=True)
    src_s = src[order]
    dst_s = dst[order]
    iE = jnp.arange(E, dtype=jnp.int32)
    occ = iE - jnp.searchsorted(dst_s, dst_s, side="left").astype(jnp.int32)
    occ = jnp.minimum(occ, KMAX - 1)
    wk = dst_s // RPW
    seg = wk * KMAX + occ
    key = seg * NPAD + dst_s
    ord2 = jnp.argsort(key)
    src_l, dst_l, seg_l = src_s[ord2], dst_s[ord2], seg[ord2]
    cnts = jnp.bincount(seg, length=NW * KMAX).astype(jnp.int32)
    padded = ((cnts + BLK - 1) // BLK) * BLK
    offs = (jnp.cumsum(padded.reshape(NW, KMAX), axis=1)
            - padded.reshape(NW, KMAX)).reshape(-1)
    pos_in = iE - jnp.searchsorted(seg_l, seg_l, side="left").astype(jnp.int32)
    slot = (seg_l // KMAX) * CAP + offs[seg_l] + pos_in
    all_slots = jnp.arange(NW * CAP, dtype=jnp.int32)
    pad_src = N + (all_slots % (NEXT - N))
    pad_dst = LROWS + (all_slots % (LPAD - LROWS))
    ldst = dst_l - (wk[ord2] - wk[ord2] // 2) * RPW
    srcs = pad_src.at[slot].set(src_l).reshape(NW, CAPB, BLK)
    dsts = pad_dst.at[slot].set(ldst).reshape(NW, CAPB, BLK)

    pid = jnp.arange(DEPAD - E, dtype=jnp.int32)
    dsts_deg = jnp.concatenate([dst, N + pid % (NPAD - N)]).reshape(
        NW, DBLKS, BLK)

    zrow = jnp.zeros((RPW, H), jnp.float32)
    zvec = jnp.zeros((NPAD // NS,), jnp.float32)
    ones = jnp.ones((BLK,), jnp.float32)
    b1r, b2r, b3r = (b.reshape(1, H) for b in (b1, b2, b3))
    gr, ber = gamma.reshape(1, H), beta.reshape(1, H)
    W1s = W1[: x.shape[1]]

    _agg_sc, _deg_sc = _sc_kernels()

    degp = _deg_sc(dsts_deg, ones, zvec)
    deg = degp[0, :N] + degp[1, :N]
    r = (1.0 / jnp.maximum(deg, 1.0)).reshape(N, 1)

    def stats1(xin, W, b):
        h = jax.nn.relu(xin @ W + b)
        m = h.mean(axis=0)
        v = ((h - m) ** 2).mean(axis=0)
        return m.reshape(1, H), v.reshape(1, H)

    def stats_mid(a, W, b):
        h = jax.nn.relu((a[:N] * r) @ W + b)
        m = h.mean(axis=0)
        v = ((h - m) ** 2).mean(axis=0)
        return m.reshape(1, H), v.reshape(1, H)

    hp1 = _pcall(_stage1, N)(x, W1s, b1r)
    m1, v1 = stats1(x, W1s, b1)
    h1 = _pcall(_norm_pad, NEXT)(hp1, m1, v1, gr, ber)
    a1 = _agg_sc(h1, srcs, dsts, zrow)

    hp2 = _pcall(_stage_mid, N)(a1, r, W2, b2r)
    m2, v2 = stats_mid(a1, W2, b2)
    h2 = _pcall(_norm_pad, NEXT)(hp2, m2, v2, gr, ber)
    a2 = _agg_sc(h2, srcs, dsts, zrow)

    hp3 = _pcall(_stage_mid, N)(a2, r, W3, b3r)
    m3, v3 = stats_mid(a2, W3, b3)
    h3 = _pcall(_norm, N)(hp3, m3, v3, gr, ber)

    pooled = h3.mean(axis=0, keepdims=True)
    regime_logits = pooled @ Wr + br
    risk = jax.nn.sigmoid(pooled @ Wk + bk)
    return (regime_logits, risk, h3)
